# baseline (speedup 1.0000x reference)
import functools

import jax
import jax.numpy as jnp
from jax import lax
from jax.experimental import pallas as pl
from jax.experimental.pallas import tpu as pltpu
from jax.experimental.pallas import tpu_sc as plsc

D = 64
HIDDEN = 128
OUT_DIM = 32


@functools.lru_cache(maxsize=None)
def _make_gather(B, D_):
    info = plsc.get_sparse_core_info()
    NC, NS = info.num_cores, info.num_subcores
    NW = NC * NS
    b_per_w = B // NW
    mesh = plsc.VectorSubcoreMesh(core_axis_name="c", subcore_axis_name="s")

    @functools.partial(
        pl.kernel,
        mesh=mesh,
        out_type=jax.ShapeDtypeStruct((B * D_,), jnp.float32),
        scratch_types=[
            pltpu.VMEM((b_per_w,), jnp.int32),
            pltpu.VMEM((b_per_w * D_,), jnp.float32),
            pltpu.SemaphoreType.DMA,
        ],
    )
    def gather_k(table_hbm, idx_hbm, out_hbm, idx_v, rows_v, sem):
        wid = lax.axis_index("s") * NC + lax.axis_index("c")
        base = wid * b_per_w
        pltpu.sync_copy(idx_hbm.at[pl.ds(base, b_per_w)], idx_v)

        def body(j, carry):
            v = idx_v[pl.ds(j * 16, 16)]
            off = pl.multiple_of(j * (16 * D_), 16 * D_)
            for k in range(16):
                r = v[k]
                src = pl.multiple_of(r * D_, D_)
                pltpu.async_copy(
                    table_hbm.at[pl.ds(src, D_)],
                    rows_v.at[pl.ds(off + k * D_, D_)],
                    sem,
                )
            return carry

        lax.fori_loop(0, b_per_w // 16, body, 0)
        pltpu.make_async_copy(
            out_hbm.at[pl.ds(base * D_, b_per_w * D_)], rows_v, sem
        ).wait()
        pltpu.sync_copy(rows_v, out_hbm.at[pl.ds(base * D_, b_per_w * D_)])

    return gather_k


def _mlp_body(x_ref, w1_ref, b1_ref, w2_ref, b2_ref, o_ref):
    x = x_ref[...]
    h = jnp.dot(x, w1_ref[...], preferred_element_type=jnp.float32)
    h = jnp.maximum(h + b1_ref[...], 0.0)
    o = jnp.dot(h, w2_ref[...], preferred_element_type=jnp.float32)
    o_ref[...] = o + b2_ref[...]


def _mlp(x, W1, b1, W2, b2, block_b=2048):
    B = x.shape[0]
    grid = (B // block_b,)
    return pl.pallas_call(
        _mlp_body,
        grid=grid,
        in_specs=[
            pl.BlockSpec((block_b, D), lambda i: (i, 0)),
            pl.BlockSpec((D, HIDDEN), lambda i: (0, 0)),
            pl.BlockSpec((1, HIDDEN), lambda i: (0, 0)),
            pl.BlockSpec((HIDDEN, OUT_DIM), lambda i: (0, 0)),
            pl.BlockSpec((1, OUT_DIM), lambda i: (0, 0)),
        ],
        out_specs=pl.BlockSpec((block_b, OUT_DIM), lambda i: (i, 0)),
        out_shape=jax.ShapeDtypeStruct((B, OUT_DIM), jnp.float32),
    )(x, W1, b1.reshape(1, HIDDEN), W2, b2.reshape(1, OUT_DIM))


@jax.jit
def kernel(gene_ids, emb, W1, b1, W2, b2):
    idx = gene_ids.astype(jnp.int32)
    B = idx.shape[0]
    x_flat = _make_gather(B, emb.shape[1])(emb.reshape(-1), idx)
    x = x_flat.reshape(B, emb.shape[1])
    return _mlp(x, W1, b1, W2, b2)


# speedup vs baseline: 1.0077x; 1.0077x over previous
import functools

import jax
import jax.numpy as jnp
from jax import lax
from jax.experimental import pallas as pl
from jax.experimental.pallas import tpu as pltpu
from jax.experimental.pallas import tpu_sc as plsc

D = 64
WD = 128
HIDDEN = 128
OUT_DIM = 32


@functools.lru_cache(maxsize=None)
def _make_gather(B):
    info = plsc.get_sparse_core_info()
    NC, NS = info.num_cores, info.num_subcores
    NW = NC * NS
    b_per_w = B // NW
    mesh = plsc.VectorSubcoreMesh(core_axis_name="c", subcore_axis_name="s")

    @functools.partial(
        pl.kernel,
        mesh=mesh,
        out_type=jax.ShapeDtypeStruct((B, WD), jnp.float32),
        scratch_types=[
            pltpu.VMEM((b_per_w,), jnp.int32),
            pltpu.VMEM((b_per_w,), jnp.int32),
            pltpu.VMEM((b_per_w, WD), jnp.float32),
            pltpu.SemaphoreType.DMA,
        ],
    )
    def gather_k(table_hbm, idx_hbm, out_hbm, idx_v, idx2_v, rows_v, sem):
        wid = lax.axis_index("s") * NC + lax.axis_index("c")
        base = wid * b_per_w
        pltpu.sync_copy(idx_hbm.at[pl.ds(base, b_per_w)], idx_v)

        def body(j, carry):
            idx2_v[pl.ds(j * 16, 16)] = idx_v[pl.ds(j * 16, 16)] >> 1
            return carry

        lax.fori_loop(0, b_per_w // 16, body, 0)
        pltpu.async_copy(table_hbm.at[idx2_v], rows_v, sem).wait()
        pltpu.sync_copy(rows_v, out_hbm.at[pl.ds(base, b_per_w)])

    return gather_k


def _mlp_body(x_ref, p_ref, w1_ref, b1_ref, w2_ref, b2_ref, o_ref):
    x = x_ref[...]
    p = p_ref[...]
    col = lax.broadcasted_iota(jnp.int32, x.shape, 1)
    keep = (col < D) == (p == 0)
    xm = jnp.where(keep, x, 0.0)
    h = jnp.dot(xm, w1_ref[...], preferred_element_type=jnp.float32)
    h = jnp.maximum(h + b1_ref[...], 0.0)
    o = jnp.dot(h, w2_ref[...], preferred_element_type=jnp.float32)
    o_ref[...] = o + b2_ref[...]


def _mlp(x, parity, W1s, b1, W2, b2, block_b=2048):
    B = x.shape[0]
    grid = (B // block_b,)
    return pl.pallas_call(
        _mlp_body,
        grid=grid,
        in_specs=[
            pl.BlockSpec((block_b, WD), lambda i: (i, 0)),
            pl.BlockSpec((block_b, 1), lambda i: (i, 0)),
            pl.BlockSpec((WD, HIDDEN), lambda i: (0, 0)),
            pl.BlockSpec((1, HIDDEN), lambda i: (0, 0)),
            pl.BlockSpec((HIDDEN, OUT_DIM), lambda i: (0, 0)),
            pl.BlockSpec((1, OUT_DIM), lambda i: (0, 0)),
        ],
        out_specs=pl.BlockSpec((block_b, OUT_DIM), lambda i: (i, 0)),
        out_shape=jax.ShapeDtypeStruct((B, OUT_DIM), jnp.float32),
    )(x, parity, W1s, b1.reshape(1, HIDDEN), W2, b2.reshape(1, OUT_DIM))


@jax.jit
def kernel(gene_ids, emb, W1, b1, W2, b2):
    idx = gene_ids.astype(jnp.int32)
    B = idx.shape[0]
    wide = emb.reshape(emb.shape[0] // 2, WD)
    x_wide = _make_gather(B)(wide, idx)
    parity = (idx & 1).reshape(B, 1)
    W1s = jnp.concatenate([W1, W1], axis=0)
    return _mlp(x_wide, parity, W1s, b1, W2, b2)


# speedup vs baseline: 1.3628x; 1.3525x over previous
import functools

import jax
import jax.numpy as jnp
from jax import lax
from jax.experimental import pallas as pl
from jax.experimental.pallas import tpu as pltpu
from jax.experimental.pallas import tpu_sc as plsc

D = 64
HIDDEN = 128
OUT_DIM = 32
V = 100000
Q = 4
VQP = 25088
BV = 6272


def _precompute_body(e0, e1, e2, e3, w1, b1, w2, b2, o_ref):
    dims = (((0,), (0,)), ((), ()))
    for k, e in enumerate((e0, e1, e2, e3)):
        h = lax.dot_general(e[...], w1[...], dims,
                            preferred_element_type=jnp.float32)
        h = jnp.maximum(h + b1[...], 0.0)
        o = jnp.dot(h, w2[...], preferred_element_type=jnp.float32)
        o_ref[:, 32 * k:32 * (k + 1)] = o + b2[...]


def _precompute(emb_t, W1, b1, W2, b2):
    grid = (VQP // BV,)
    nb = VQP // BV
    e_spec = lambda k: pl.BlockSpec((D, BV), lambda i, k=k: (0, nb * k + i))
    return pl.pallas_call(
        _precompute_body,
        grid=grid,
        in_specs=[
            e_spec(0), e_spec(1), e_spec(2), e_spec(3),
            pl.BlockSpec((D, HIDDEN), lambda i: (0, 0)),
            pl.BlockSpec((1, HIDDEN), lambda i: (0, 0)),
            pl.BlockSpec((HIDDEN, OUT_DIM), lambda i: (0, 0)),
            pl.BlockSpec((1, OUT_DIM), lambda i: (0, 0)),
        ],
        out_specs=pl.BlockSpec((BV, HIDDEN), lambda i: (i, 0)),
        out_shape=jax.ShapeDtypeStruct((VQP, HIDDEN), jnp.float32),
    )(emb_t, emb_t, emb_t, emb_t, W1,
      b1.reshape(1, HIDDEN), W2, b2.reshape(1, OUT_DIM))


@functools.lru_cache(maxsize=None)
def _make_gather(B):
    info = plsc.get_sparse_core_info()
    NC, NS = info.num_cores, info.num_subcores
    NW = NC * NS
    b_per_w = B // NW
    mesh = plsc.VectorSubcoreMesh(core_axis_name="c", subcore_axis_name="s")

    @functools.partial(
        pl.kernel,
        mesh=mesh,
        out_type=jax.ShapeDtypeStruct((B, HIDDEN), jnp.float32),
        scratch_types=[
            pltpu.VMEM((b_per_w,), jnp.int32),
            pltpu.VMEM((b_per_w,), jnp.int32),
            pltpu.VMEM((b_per_w, HIDDEN), jnp.float32),
            pltpu.SemaphoreType.DMA,
        ],
    )
    def gather_k(table_hbm, idx_hbm, out_hbm, idx_v, idx2_v, rows_v, sem):
        wid = lax.axis_index("s") * NC + lax.axis_index("c")
        base = wid * b_per_w
        pltpu.sync_copy(idx_hbm.at[pl.ds(base, b_per_w)], idx_v)

        def body(j, carry):
            sl = pl.ds(j * 16, 16)
            idx2_v[sl] = lax.rem(idx_v[sl], VQP)
            return carry

        lax.fori_loop(0, b_per_w // 16, body, 0)
        pltpu.async_copy(table_hbm.at[idx2_v], rows_v, sem).wait()
        pltpu.sync_copy(rows_v, out_hbm.at[pl.ds(base, b_per_w)])

    return gather_k


def _select_body(w_ref, q_ref, o_ref):
    w = w_ref[...]
    q = q_ref[...]
    o_ref[...] = jnp.where(
        q == 0, w[:, 0:32],
        jnp.where(q == 1, w[:, 32:64],
                  jnp.where(q == 2, w[:, 64:96], w[:, 96:128])))


def _select(wide, q, block_b=2048):
    B = wide.shape[0]
    grid = (B // block_b,)
    return pl.pallas_call(
        _select_body,
        grid=grid,
        in_specs=[
            pl.BlockSpec((block_b, HIDDEN), lambda i: (i, 0)),
            pl.BlockSpec((block_b, 1), lambda i: (i, 0)),
        ],
        out_specs=pl.BlockSpec((block_b, OUT_DIM), lambda i: (i, 0)),
        out_shape=jax.ShapeDtypeStruct((B, OUT_DIM), jnp.float32),
    )(wide, q)


@jax.jit
def kernel(gene_ids, emb, W1, b1, W2, b2):
    idx = gene_ids.astype(jnp.int32)
    B = idx.shape[0]
    emb_t = jnp.transpose(emb)
    table = _precompute(emb_t, W1, b1, W2, b2)
    wide = _make_gather(B)(table, idx)
    q = (idx // VQP).reshape(B, 1)
    return _select(wide, q)


# speedup vs baseline: 1.4997x; 1.1004x over previous
import functools

import jax
import jax.numpy as jnp
from jax import lax
from jax.experimental import pallas as pl
from jax.experimental.pallas import tpu as pltpu
from jax.experimental.pallas import tpu_sc as plsc

D = 64
HIDDEN = 128
OUT_DIM = 32
V = 100000
VQP = 25088
BV = 6272


def _precompute_body(e0, e1, e2, e3, w1t, b1t, w2t, b2t, o_ref):
    for k, e in enumerate((e0, e1, e2, e3)):
        ht = jnp.dot(w1t[...], e[...], preferred_element_type=jnp.float32)
        ht = jnp.maximum(ht + b1t[...], 0.0)
        ot = jnp.dot(w2t[...], ht, preferred_element_type=jnp.float32)
        o_ref[:, 32 * k:32 * (k + 1)] = jnp.transpose(ot + b2t[...])


def _precompute(emb_t, W1t, b1t, W2t, b2t):
    grid = (VQP // BV,)
    nb = VQP // BV
    e_spec = lambda k: pl.BlockSpec((D, BV), lambda i, k=k: (0, nb * k + i))
    return pl.pallas_call(
        _precompute_body,
        grid=grid,
        in_specs=[
            e_spec(0), e_spec(1), e_spec(2), e_spec(3),
            pl.BlockSpec((HIDDEN, D), lambda i: (0, 0)),
            pl.BlockSpec((HIDDEN, 1), lambda i: (0, 0)),
            pl.BlockSpec((OUT_DIM, HIDDEN), lambda i: (0, 0)),
            pl.BlockSpec((OUT_DIM, 1), lambda i: (0, 0)),
        ],
        out_specs=pl.BlockSpec((BV, HIDDEN), lambda i: (i, 0)),
        out_shape=jax.ShapeDtypeStruct((VQP, HIDDEN), jnp.float32),
    )(emb_t, emb_t, emb_t, emb_t, W1t, b1t, W2t, b2t)


@functools.lru_cache(maxsize=None)
def _make_gather(B):
    info = plsc.get_sparse_core_info()
    NC, NS = info.num_cores, info.num_subcores
    NW = NC * NS
    b_per_w = B // NW
    mesh = plsc.VectorSubcoreMesh(core_axis_name="c", subcore_axis_name="s")

    @functools.partial(
        pl.kernel,
        mesh=mesh,
        out_type=jax.ShapeDtypeStruct((OUT_DIM, B), jnp.float32),
        scratch_types=[
            pltpu.VMEM((b_per_w,), jnp.int32),
            pltpu.VMEM((b_per_w,), jnp.int32),
            pltpu.VMEM((b_per_w, HIDDEN), jnp.float32),
            pltpu.VMEM((OUT_DIM, b_per_w), jnp.float32),
            pltpu.SemaphoreType.DMA,
        ],
        compiler_params=pltpu.CompilerParams(needs_layout_passes=False),
    )
    def gather_k(table_hbm, idx_hbm, outT_hbm, idx_v, idx2_v, rows_v, xt_v,
                 sem):
        wid = lax.axis_index("s") * NC + lax.axis_index("c")
        base = wid * b_per_w
        pltpu.sync_copy(idx_hbm.at[pl.ds(base, b_per_w)], idx_v)

        def mod_body(j, carry):
            sl = pl.ds(j * 16, 16)
            idx2_v[sl] = lax.rem(idx_v[sl], VQP)
            return carry

        lax.fori_loop(0, b_per_w // 16, mod_body, 0)
        pltpu.async_copy(table_hbm.at[idx2_v], rows_v, sem).wait()

        def sel_body(g, carry):
            sl = pl.ds(g * 16, 16)
            iv = idx_v[sl]
            q = lax.shift_right_logical(
                lax.shift_right_logical(iv, 9) * 1338, 16)
            q32 = q * 32
            rvec = lax.iota(jnp.int32, 16) + g * 16
            for d in range(OUT_DIM):
                xt_v[d, sl] = plsc.load_gather(rows_v, [rvec, q32 + d])
            return carry

        lax.fori_loop(0, b_per_w // 16, sel_body, 0)
        pltpu.sync_copy(xt_v, outT_hbm.at[:, pl.ds(base, b_per_w)])

    return gather_k


@jax.jit
def kernel(gene_ids, emb, W1, b1, W2, b2):
    idx = gene_ids.astype(jnp.int32)
    B = idx.shape[0]
    emb_t = jnp.transpose(emb)
    table = _precompute(emb_t, W1.T, b1.reshape(HIDDEN, 1),
                        W2.T, b2.reshape(OUT_DIM, 1))
    outT = _make_gather(B)(table, idx)
    return jnp.transpose(outT)


# speedup vs baseline: 1.6793x; 1.1198x over previous
import functools

import jax
import jax.numpy as jnp
from jax import lax
from jax.experimental import pallas as pl
from jax.experimental.pallas import tpu as pltpu
from jax.experimental.pallas import tpu_sc as plsc

D = 64
HIDDEN = 128
OUT_DIM = 32
V = 100000
VQP = 25088
BV = 6272


def _precompute_body(e0, e1, e2, e3, w1, b1, w2, b2, o_ref):
    for k, e in enumerate((e0, e1, e2, e3)):
        et = jnp.transpose(e[...])
        h = jnp.dot(et, w1[...], preferred_element_type=jnp.float32)
        h = jnp.maximum(h + b1[...], 0.0)
        o = jnp.dot(h, w2[...], preferred_element_type=jnp.float32)
        o_ref[:, 32 * k:32 * (k + 1)] = o + b2[...]


def _precompute(emb_t, W1t, b1t, W2t, b2t):
    grid = (VQP // BV,)
    nb = VQP // BV
    e_spec = lambda k: pl.BlockSpec((D, BV), lambda i, k=k: (0, nb * k + i))
    return pl.pallas_call(
        _precompute_body,
        grid=grid,
        in_specs=[
            e_spec(0), e_spec(1), e_spec(2), e_spec(3),
            pl.BlockSpec((D, HIDDEN), lambda i: (0, 0)),
            pl.BlockSpec((1, HIDDEN), lambda i: (0, 0)),
            pl.BlockSpec((HIDDEN, OUT_DIM), lambda i: (0, 0)),
            pl.BlockSpec((1, OUT_DIM), lambda i: (0, 0)),
        ],
        out_specs=pl.BlockSpec((BV, HIDDEN), lambda i: (i, 0)),
        out_shape=jax.ShapeDtypeStruct((VQP, HIDDEN), jnp.float32),
    )(emb_t, emb_t, emb_t, emb_t, W1t, b1t, W2t, b2t)


@functools.lru_cache(maxsize=None)
def _make_gather(B):
    info = plsc.get_sparse_core_info()
    NC, NS = info.num_cores, info.num_subcores
    NW = NC * NS
    b_per_w = B // NW
    mesh = plsc.VectorSubcoreMesh(core_axis_name="c", subcore_axis_name="s")

    @functools.partial(
        pl.kernel,
        mesh=mesh,
        out_type=jax.ShapeDtypeStruct((OUT_DIM, B), jnp.float32),
        scratch_types=[
            pltpu.VMEM((b_per_w,), jnp.int32),
            pltpu.VMEM((b_per_w,), jnp.int32),
            pltpu.VMEM((b_per_w, HIDDEN), jnp.float32),
            pltpu.VMEM((OUT_DIM, b_per_w), jnp.float32),
            pltpu.SemaphoreType.DMA,
        ],
        compiler_params=pltpu.CompilerParams(needs_layout_passes=False),
    )
    def gather_k(table_hbm, idx_hbm, outT_hbm, idx_v, idx2_v, rows_v, xt_v,
                 sem):
        wid = lax.axis_index("s") * NC + lax.axis_index("c")
        base = wid * b_per_w
        pltpu.sync_copy(idx_hbm.at[pl.ds(base, b_per_w)], idx_v)

        def mod_body(j, carry):
            sl = pl.ds(j * 16, 16)
            idx2_v[sl] = lax.rem(idx_v[sl], VQP)
            return carry

        lax.fori_loop(0, b_per_w // 16, mod_body, 0)
        pltpu.async_copy(table_hbm.at[idx2_v], rows_v, sem).wait()

        def sel_body(g, carry):
            sl = pl.ds(g * 16, 16)
            iv = idx_v[sl]
            q = lax.shift_right_logical(
                lax.shift_right_logical(iv, 9) * 1338, 16)
            q32 = q * 32
            rvec = lax.iota(jnp.int32, 16) + g * 16
            for d in range(OUT_DIM):
                xt_v[d, sl] = plsc.load_gather(rows_v, [rvec, q32 + d])
            return carry

        lax.fori_loop(0, b_per_w // 16, sel_body, 0)
        pltpu.sync_copy(xt_v, outT_hbm.at[:, pl.ds(base, b_per_w)])

    return gather_k


@jax.jit
def kernel(gene_ids, emb, W1, b1, W2, b2):
    idx = gene_ids.astype(jnp.int32)
    B = idx.shape[0]
    emb_t = jnp.transpose(emb)
    table = _precompute(emb_t, W1, b1.reshape(1, HIDDEN),
                        W2, b2.reshape(1, OUT_DIM))
    outT = _make_gather(B)(table, idx)
    return jnp.transpose(outT)
